# Initial kernel scaffold; baseline (speedup 1.0000x reference)
import jax
import jax.numpy as jnp
from jax.experimental import pallas as pl


def kernel(x, edge_index, W, b):
    raise NotImplementedError("write your pallas kernel here")


# speedup vs baseline: 15.8496x; 15.8496x over previous
import functools

import jax
import jax.numpy as jnp
from jax import lax
from jax.experimental import pallas as pl
from jax.experimental.pallas import tpu as pltpu
from jax.experimental.pallas import tpu_sc as plsc

NC = 2
NS = 16
NW = NC * NS
LANES = 16
CHUNK = 128
BN = 2048


def _deg_kernel_make(n_pad, ept):
    cpt = ept // CHUNK
    mesh = plsc.VectorSubcoreMesh(
        core_axis_name="c", subcore_axis_name="s",
        num_cores=NC, num_subcores=NS)

    @functools.partial(
        pl.kernel,
        out_type=jax.ShapeDtypeStruct((NW, n_pad), jnp.float32),
        mesh=mesh,
        scratch_types=[
            pltpu.VMEM((n_pad,), jnp.float32),
            pltpu.VMEM((CHUNK,), jnp.int32),
        ],
        compiler_params=pltpu.CompilerParams(needs_layout_passes=False),
    )
    def deg_kernel(dst_hbm, out_hbm, hist_v, idx_v):
        cid = lax.axis_index("c")
        sid = lax.axis_index("s")
        wid = sid * NC + cid

        zero = jnp.zeros((LANES,), jnp.float32)

        def zbody(i, carry):
            hist_v[pl.ds(i * LANES, LANES)] = zero
            return carry

        lax.fori_loop(0, n_pad // LANES, zbody, 0)

        ones = jnp.ones((LANES,), jnp.float32)

        def chunk_body(ci, carry):
            base = wid * ept + ci * CHUNK
            pltpu.sync_copy(dst_hbm.at[pl.ds(base, CHUNK)], idx_v)

            def sub(j, c2):
                idx = idx_v[pl.ds(j * LANES, LANES)]
                plsc.addupdate_scatter(hist_v, [idx], ones)
                return c2

            lax.fori_loop(0, CHUNK // LANES, sub, 0)
            return carry

        lax.fori_loop(0, cpt, chunk_body, 0)
        pltpu.sync_copy(hist_v, out_hbm.at[wid])

    return deg_kernel


def _agg_kernel_make(n_pad, ept, h):
    cpt = ept // CHUNK
    rpt = n_pad // NS
    mesh = plsc.VectorSubcoreMesh(
        core_axis_name="c", subcore_axis_name="s",
        num_cores=NC, num_subcores=NS)

    @functools.partial(
        pl.kernel,
        out_type=jax.ShapeDtypeStruct((NC, n_pad, h), jnp.float32),
        mesh=mesh,
        scratch_types=[
            pltpu.VMEM((CHUNK,), jnp.int32),
            pltpu.VMEM((CHUNK,), jnp.int32),
            pltpu.VMEM((CHUNK, h), jnp.float32),
            pltpu.VMEM_SHARED((n_pad, h), jnp.float32),
            pltpu.SemaphoreType.DMA,
        ],
        compiler_params=pltpu.CompilerParams(needs_layout_passes=False),
    )
    def agg_kernel(y_hbm, src_hbm, dst_hbm, zeros_hbm, out_hbm,
                   sidx_v, didx_v, rows_v, agg_s, sem):
        cid = lax.axis_index("c")
        sid = lax.axis_index("s")
        wid = sid * NC + cid

        pltpu.sync_copy(zeros_hbm, rows_v)
        for k in range(rpt // CHUNK):
            pltpu.sync_copy(rows_v,
                            agg_s.at[pl.ds(sid * rpt + k * CHUNK, CHUNK)])
        plsc.subcore_barrier()

        def chunk_body(ci, carry):
            base = wid * ept + ci * CHUNK
            pltpu.sync_copy(src_hbm.at[pl.ds(base, CHUNK)], sidx_v)
            pltpu.sync_copy(dst_hbm.at[pl.ds(base, CHUNK)], didx_v)
            pltpu.async_copy(y_hbm.at[sidx_v], rows_v, sem).wait()
            pltpu.sync_copy(rows_v, agg_s.at[didx_v], add=True)
            return carry

        lax.fori_loop(0, cpt, chunk_body, 0)
        plsc.subcore_barrier()

        for k in range(rpt // CHUNK):
            r0 = sid * rpt + k * CHUNK
            pltpu.sync_copy(agg_s.at[pl.ds(r0, CHUNK)], rows_v)
            pltpu.sync_copy(rows_v, out_hbm.at[cid, pl.ds(r0, CHUNK)])

    return agg_kernel


def _scale_kernel_make(n_pad, d, h):
    grid = (n_pad // BN,)

    def body(x_ref, w_ref, p_ref, y_ref):
        xw = jnp.dot(x_ref[:], w_ref[:],
                     preferred_element_type=jnp.float32,
                     precision=lax.Precision.HIGHEST)
        ones = jnp.ones((NW, 1), jnp.float32)
        deg = lax.dot_general(p_ref[:], ones, (((0,), (0,)), ((), ())),
                              precision=lax.Precision.HIGHEST) + 1.0
        y_ref[:] = xw * lax.rsqrt(deg)

    return pl.pallas_call(
        body,
        grid=grid,
        in_specs=[
            pl.BlockSpec((BN, d), lambda i: (i, 0)),
            pl.BlockSpec((d, h), lambda i: (0, 0)),
            pl.BlockSpec((NW, BN), lambda i: (0, i)),
        ],
        out_specs=pl.BlockSpec((BN, h), lambda i: (i, 0)),
        out_shape=jax.ShapeDtypeStruct((n_pad, h), jnp.float32),
    )


def _merge_kernel_make(n_pad, h):
    grid = (n_pad // BN,)

    def body(pc_ref, y_ref, p_ref, b_ref, o_ref):
        ones = jnp.ones((NW, 1), jnp.float32)
        deg = lax.dot_general(p_ref[:], ones, (((0,), (0,)), ((), ())),
                              precision=lax.Precision.HIGHEST) + 1.0
        dis = lax.rsqrt(deg)
        s = pc_ref[0] + pc_ref[1] + y_ref[:]
        o_ref[:] = jnp.maximum(s * dis + b_ref[:], 0.0)

    return pl.pallas_call(
        body,
        grid=grid,
        in_specs=[
            pl.BlockSpec((NC, BN, h), lambda i: (0, i, 0)),
            pl.BlockSpec((BN, h), lambda i: (i, 0)),
            pl.BlockSpec((NW, BN), lambda i: (0, i)),
            pl.BlockSpec((1, h), lambda i: (0, 0)),
        ],
        out_specs=pl.BlockSpec((BN, h), lambda i: (i, 0)),
        out_shape=jax.ShapeDtypeStruct((n_pad, h), jnp.float32),
    )


def kernel(x, edge_index, W, b):
    n, d = x.shape
    h = W.shape[1]
    e = edge_index.shape[1]

    ept = -(-e // (NW * CHUNK)) * CHUNK
    e_pad = ept * NW
    n_pad = -(-(n + 1) // (NS * CHUNK)) * (NS * CHUNK)
    n_pad = -(-n_pad // BN) * BN

    pad = jnp.full((e_pad - e,), n, dtype=edge_index.dtype)
    src = jnp.concatenate([edge_index[0], pad])
    dst = jnp.concatenate([edge_index[1], pad])
    x_pad = jnp.zeros((n_pad, d), x.dtype).at[:n].set(x)
    zeros_chunk = jnp.zeros((CHUNK, h), jnp.float32)

    partials = _deg_kernel_make(n_pad, ept)(dst)
    y = _scale_kernel_make(n_pad, d, h)(x_pad, W, partials)
    pc = _agg_kernel_make(n_pad, ept, h)(y, src, dst, zeros_chunk)
    out = _merge_kernel_make(n_pad, h)(pc, y, partials, b[None, :])
    return out[:n]


# speedup vs baseline: 17.4524x; 1.1011x over previous
import functools

import jax
import jax.numpy as jnp
from jax import lax
from jax.experimental import pallas as pl
from jax.experimental.pallas import tpu as pltpu
from jax.experimental.pallas import tpu_sc as plsc

NC = 2
NS = 16
NW = NC * NS
LANES = 16
CHUNK = 128


def _pad_kernel_make(e, e_pad, n, dtype):
    def body(ei_ref, src_ref, dst_ref):
        fill = jnp.full((e_pad - e,), n, dtype)
        src_ref[pl.ds(0, e)] = ei_ref[0]
        src_ref[pl.ds(e, e_pad - e)] = fill
        dst_ref[pl.ds(0, e)] = ei_ref[1]
        dst_ref[pl.ds(e, e_pad - e)] = fill

    return pl.pallas_call(
        body,
        out_shape=(jax.ShapeDtypeStruct((e_pad,), dtype),
                   jax.ShapeDtypeStruct((e_pad,), dtype)),
    )


def _deg_kernel_make(n_acc, cpt):
    mesh = plsc.VectorSubcoreMesh(
        core_axis_name="c", subcore_axis_name="s",
        num_cores=NC, num_subcores=NS)

    @functools.partial(
        pl.kernel,
        out_type=jax.ShapeDtypeStruct((NW, n_acc), jnp.float32),
        mesh=mesh,
        scratch_types=[
            pltpu.VMEM((n_acc,), jnp.float32),
            pltpu.VMEM((cpt, CHUNK), jnp.int32),
        ],
        compiler_params=pltpu.CompilerParams(needs_layout_passes=False),
    )
    def deg_kernel(dst_hbm, zeros_hbm, out_hbm, hist_v, idx_v):
        cid = lax.axis_index("c")
        sid = lax.axis_index("s")
        wid = sid * NC + cid

        pltpu.sync_copy(zeros_hbm, hist_v)
        pltpu.sync_copy(dst_hbm.at[wid], idx_v)

        ones = jnp.ones((LANES,), jnp.float32)

        def body(ci, carry):
            for j in range(CHUNK // LANES):
                idx = idx_v[ci, pl.ds(j * LANES, LANES)]
                plsc.addupdate_scatter(hist_v, [idx], ones)
            return carry

        lax.fori_loop(0, cpt, body, 0)
        pltpu.sync_copy(hist_v, out_hbm.at[wid])

    return deg_kernel


def _agg_kernel_make(n_acc, cpt, h):
    ept = cpt * CHUNK
    rpt = n_acc // NS
    mesh = plsc.VectorSubcoreMesh(
        core_axis_name="c", subcore_axis_name="s",
        num_cores=NC, num_subcores=NS)

    @functools.partial(
        pl.kernel,
        out_type=jax.ShapeDtypeStruct((NC, n_acc, h), jnp.float32),
        mesh=mesh,
        scratch_types=[
            pltpu.VMEM((CHUNK,), jnp.int32),
            pltpu.VMEM((CHUNK,), jnp.int32),
            pltpu.VMEM((CHUNK,), jnp.int32),
            pltpu.VMEM((CHUNK,), jnp.int32),
            pltpu.VMEM((CHUNK, h), jnp.float32),
            pltpu.VMEM((CHUNK, h), jnp.float32),
            pltpu.VMEM_SHARED((n_acc, h), jnp.float32),
            pltpu.SemaphoreType.DMA,
            pltpu.SemaphoreType.DMA,
        ],
        compiler_params=pltpu.CompilerParams(needs_layout_passes=False),
    )
    def agg_kernel(y_hbm, src_hbm, dst_hbm, zeros_hbm, out_hbm,
                   sidx0, sidx1, didx0, didx1, rows0, rows1,
                   agg_s, sem_i, sem_g):
        cid = lax.axis_index("c")
        sid = lax.axis_index("s")
        wid = sid * NC + cid
        sidx = (sidx0, sidx1)
        didx = (didx0, didx1)
        rows = (rows0, rows1)

        def idx_copies(c, b):
            base = wid * ept + c * CHUNK
            return (
                pltpu.make_async_copy(
                    src_hbm.at[pl.ds(base, CHUNK)], sidx[b], sem_i),
                pltpu.make_async_copy(
                    dst_hbm.at[pl.ds(base, CHUNK)], didx[b], sem_i),
            )

        def gather(b):
            return pltpu.make_async_copy(
                y_hbm.at[sidx[b]], rows[b], sem_g)

        pltpu.sync_copy(zeros_hbm, rows0)
        for k in range(rpt // CHUNK):
            pltpu.sync_copy(rows0,
                            agg_s.at[pl.ds(sid * rpt + k * CHUNK, CHUNK)])
        plsc.subcore_barrier()

        for cp in idx_copies(0, 0):
            cp.start()

        def pair_body(g, carry):
            for b in range(2):
                c = 2 * g + b
                for cp in idx_copies(c, b):
                    cp.wait()
                gather(b).start()

                @pl.when(c > 0)
                def _():
                    gather(1 - b).wait()
                    pltpu.sync_copy(rows[1 - b],
                                    agg_s.at[didx[1 - b]], add=True)

                @pl.when(c + 1 < cpt)
                def _():
                    for cp in idx_copies(c + 1, 1 - b):
                        cp.start()
            return carry

        lax.fori_loop(0, cpt // 2, pair_body, 0)
        gather(1).wait()
        pltpu.sync_copy(rows1, agg_s.at[didx1], add=True)
        plsc.subcore_barrier()

        for k in range(rpt // CHUNK):
            r0 = sid * rpt + k * CHUNK
            pltpu.sync_copy(agg_s.at[pl.ds(r0, CHUNK)], rows0)
            pltpu.sync_copy(rows0, out_hbm.at[cid, pl.ds(r0, CHUNK)])

    return agg_kernel


def _scale_kernel_make(n, n_acc, d, h):
    def body(x_ref, w_ref, p_ref, y_ref):
        xw = jnp.dot(x_ref[:], w_ref[:],
                     preferred_element_type=jnp.float32,
                     precision=lax.Precision.HIGHEST)
        ones = jnp.ones((NW, 1), jnp.float32)
        deg = lax.dot_general(p_ref[:], ones, (((0,), (0,)), ((), ())),
                              precision=lax.Precision.HIGHEST) + 1.0
        dis = lax.rsqrt(deg)
        y_ref[pl.ds(0, n)] = xw * dis[:n]
        y_ref[pl.ds(n, n_acc - n)] = jnp.zeros((n_acc - n, h), jnp.float32)

    return pl.pallas_call(
        body,
        out_shape=jax.ShapeDtypeStruct((n_acc, h), jnp.float32),
    )


def _merge_kernel_make(n, h):
    def body(pc_ref, y_ref, p_ref, b_ref, o_ref):
        ones = jnp.ones((NW, 1), jnp.float32)
        deg = lax.dot_general(p_ref[:], ones, (((0,), (0,)), ((), ())),
                              precision=lax.Precision.HIGHEST) + 1.0
        dis = lax.rsqrt(deg)
        s = pc_ref[0, :n] + pc_ref[1, :n] + y_ref[:n]
        o_ref[:] = jnp.maximum(s * dis[:n] + b_ref[:], 0.0)

    return pl.pallas_call(
        body,
        out_shape=jax.ShapeDtypeStruct((n, h), jnp.float32),
    )


def kernel(x, edge_index, W, b):
    n, d = x.shape
    h = W.shape[1]
    e = edge_index.shape[1]

    cpt = -(-e // (NW * CHUNK))
    cpt += cpt % 2
    e_pad = cpt * CHUNK * NW
    n_acc = -(-(n + 1) // (NS * CHUNK)) * (NS * CHUNK)

    src, dst = _pad_kernel_make(e, e_pad, n, edge_index.dtype)(edge_index)
    dst3 = dst.reshape(NW, cpt, CHUNK)
    zeros_stripe = jnp.zeros((CHUNK, h), jnp.float32)
    zeros_hist = jnp.zeros((n_acc,), jnp.float32)

    partials = _deg_kernel_make(n_acc, cpt)(dst3, zeros_hist)
    y = _scale_kernel_make(n, n_acc, d, h)(x, W, partials)
    pc = _agg_kernel_make(n_acc, cpt, h)(y, src, dst, zeros_stripe)
    return _merge_kernel_make(n, h)(pc, y, partials, b[None, :])


# speedup vs baseline: 42.8472x; 2.4551x over previous
import functools

import jax
import jax.numpy as jnp
from jax import lax
from jax.experimental import pallas as pl
from jax.experimental.pallas import tpu as pltpu
from jax.experimental.pallas import tpu_sc as plsc

NC = 2
NS = 16
NW = NC * NS
LANES = 16
CHUNK = 128


def _pad_kernel_make(e, e_pad, n, n_acc, dtype):
    def body(ei_ref, fill_ref, src_ref, dst_ref):
        fill = fill_ref[:]
        src_ref[pl.ds(0, e)] = ei_ref[0]
        src_ref[pl.ds(e, e_pad - e)] = fill
        dst_ref[pl.ds(0, e)] = ei_ref[1]
        dst_ref[pl.ds(e, e_pad - e)] = fill

    return pl.pallas_call(
        body,
        out_shape=(jax.ShapeDtypeStruct((e_pad,), dtype),
                   jax.ShapeDtypeStruct((e_pad,), dtype)),
    )


def _deg_kernel_make(n_acc, cpt):
    mesh = plsc.VectorSubcoreMesh(
        core_axis_name="c", subcore_axis_name="s",
        num_cores=NC, num_subcores=NS)

    @functools.partial(
        pl.kernel,
        out_type=jax.ShapeDtypeStruct((NW, n_acc), jnp.float32),
        mesh=mesh,
        scratch_types=[
            pltpu.VMEM((n_acc,), jnp.float32),
            pltpu.VMEM((cpt, CHUNK), jnp.int32),
        ],
        compiler_params=pltpu.CompilerParams(needs_layout_passes=False),
    )
    def deg_kernel(dst_hbm, zeros_hbm, out_hbm, hist_v, idx_v):
        cid = lax.axis_index("c")
        sid = lax.axis_index("s")
        wid = sid * NC + cid

        pltpu.sync_copy(zeros_hbm, hist_v)
        pltpu.sync_copy(dst_hbm.at[wid], idx_v)

        ones = jnp.ones((LANES,), jnp.float32)

        def body(ci, carry):
            for j in range(CHUNK // LANES):
                idx = idx_v[ci, pl.ds(j * LANES, LANES)]
                plsc.addupdate_scatter(hist_v, [idx], ones)
            return carry

        lax.fori_loop(0, cpt, body, 0)
        pltpu.sync_copy(hist_v, out_hbm.at[wid])

    return deg_kernel


def _agg_kernel_make(n_acc, cpt, h):
    ept = cpt * CHUNK
    rpt = n_acc // NS
    mesh = plsc.VectorSubcoreMesh(
        core_axis_name="c", subcore_axis_name="s",
        num_cores=NC, num_subcores=NS)

    @functools.partial(
        pl.kernel,
        out_type=jax.ShapeDtypeStruct((NC, n_acc, h), jnp.float32),
        mesh=mesh,
        scratch_types=[
            pltpu.VMEM((CHUNK,), jnp.int32),
            pltpu.VMEM((CHUNK,), jnp.int32),
            pltpu.VMEM((CHUNK,), jnp.int32),
            pltpu.VMEM((CHUNK,), jnp.int32),
            pltpu.VMEM((CHUNK, h), jnp.float32),
            pltpu.VMEM((CHUNK, h), jnp.float32),
            pltpu.VMEM_SHARED((n_acc, h), jnp.float32),
            pltpu.SemaphoreType.DMA,
            pltpu.SemaphoreType.DMA,
        ],
        compiler_params=pltpu.CompilerParams(needs_layout_passes=False),
    )
    def agg_kernel(y_hbm, src_hbm, dst_hbm, zeros_hbm, out_hbm,
                   sidx0, sidx1, didx0, didx1, rows0, rows1,
                   agg_s, sem_i, sem_g):
        cid = lax.axis_index("c")
        sid = lax.axis_index("s")
        wid = sid * NC + cid
        sidx = (sidx0, sidx1)
        didx = (didx0, didx1)
        rows = (rows0, rows1)

        def idx_copies(c, b):
            base = wid * ept + c * CHUNK
            return (
                pltpu.make_async_copy(
                    src_hbm.at[pl.ds(base, CHUNK)], sidx[b], sem_i),
                pltpu.make_async_copy(
                    dst_hbm.at[pl.ds(base, CHUNK)], didx[b], sem_i),
            )

        def gather(b):
            return pltpu.make_async_copy(
                y_hbm.at[sidx[b]], rows[b], sem_g)

        pltpu.sync_copy(zeros_hbm, rows0)
        for k in range(rpt // CHUNK):
            pltpu.sync_copy(rows0,
                            agg_s.at[pl.ds(sid * rpt + k * CHUNK, CHUNK)])
        plsc.subcore_barrier()

        for cp in idx_copies(0, 0):
            cp.start()

        def pair_body(g, carry):
            for b in range(2):
                c = 2 * g + b
                for cp in idx_copies(c, b):
                    cp.wait()
                gather(b).start()

                @pl.when(c > 0)
                def _():
                    gather(1 - b).wait()
                    pltpu.sync_copy(rows[1 - b],
                                    agg_s.at[didx[1 - b]], add=True)

                @pl.when(c + 1 < cpt)
                def _():
                    for cp in idx_copies(c + 1, 1 - b):
                        cp.start()
            return carry

        lax.fori_loop(0, cpt // 2, pair_body, 0)
        gather(1).wait()
        pltpu.sync_copy(rows1, agg_s.at[didx1], add=True)
        plsc.subcore_barrier()

        for k in range(rpt // CHUNK):
            r0 = sid * rpt + k * CHUNK
            pltpu.sync_copy(agg_s.at[pl.ds(r0, CHUNK)], rows0)
            pltpu.sync_copy(rows0, out_hbm.at[cid, pl.ds(r0, CHUNK)])

    return agg_kernel


def _scale_kernel_make(n, n_acc, d, h):
    def body(x_ref, w_ref, p_ref, y_ref):
        xw = jnp.dot(x_ref[:], w_ref[:],
                     preferred_element_type=jnp.float32,
                     precision=lax.Precision.HIGHEST)
        ones = jnp.ones((NW, 1), jnp.float32)
        deg = lax.dot_general(p_ref[:], ones, (((0,), (0,)), ((), ())),
                              precision=lax.Precision.HIGHEST) + 1.0
        dis = lax.rsqrt(deg)
        y_ref[pl.ds(0, n)] = xw * dis[:n]
        y_ref[pl.ds(n, n_acc - n)] = jnp.zeros((n_acc - n, h), jnp.float32)

    return pl.pallas_call(
        body,
        out_shape=jax.ShapeDtypeStruct((n_acc, h), jnp.float32),
    )


def _merge_kernel_make(n, h):
    def body(pc_ref, y_ref, p_ref, b_ref, o_ref):
        ones = jnp.ones((NW, 1), jnp.float32)
        deg = lax.dot_general(p_ref[:], ones, (((0,), (0,)), ((), ())),
                              precision=lax.Precision.HIGHEST) + 1.0
        dis = lax.rsqrt(deg)
        s = pc_ref[0, :n] + pc_ref[1, :n] + y_ref[:n]
        o_ref[:] = jnp.maximum(s * dis[:n] + b_ref[:], 0.0)

    return pl.pallas_call(
        body,
        out_shape=jax.ShapeDtypeStruct((n, h), jnp.float32),
    )


def kernel(x, edge_index, W, b):
    n, d = x.shape
    h = W.shape[1]
    e = edge_index.shape[1]

    cpt = -(-e // (NW * CHUNK))
    cpt += cpt % 2
    e_pad = cpt * CHUNK * NW
    n_acc = -(-(n + 1) // (NS * CHUNK)) * (NS * CHUNK)

    fill = (n + jnp.arange(e_pad - e, dtype=edge_index.dtype)
            % jnp.asarray(n_acc - n, edge_index.dtype))
    src, dst = _pad_kernel_make(e, e_pad, n, n_acc,
                                edge_index.dtype)(edge_index, fill)
    dst3 = dst.reshape(NW, cpt, CHUNK)
    zeros_stripe = jnp.zeros((CHUNK, h), jnp.float32)
    zeros_hist = jnp.zeros((n_acc,), jnp.float32)

    partials = _deg_kernel_make(n_acc, cpt)(dst3, zeros_hist)
    y = _scale_kernel_make(n, n_acc, d, h)(x, W, partials)
    pc = _agg_kernel_make(n_acc, cpt, h)(y, src, dst, zeros_stripe)
    return _merge_kernel_make(n, h)(pc, y, partials, b[None, :])
